# baseline (speedup 1.0000x reference)
import functools

import jax
import jax.numpy as jnp
from jax import lax
from jax.experimental import pallas as pl
from jax.experimental.pallas import tpu as pltpu
from jax.experimental.pallas import tpu_sc as plsc

T, D, S = 16384, 2048, 16
T_TILE = 256
NSTREAM = 8
NCHUNK = T // T_TILE
NW = 32
CPW = NCHUNK // NW
NGRP = T_TILE // 16


def _tc_body(*refs):
    x_refs = refs[:NSTREAM]
    g_refs = refs[NSTREAM:2 * NSTREAM]
    wa_ref, ws_ref, batt_ref, abias_ref, bstrat_ref, out_ref = refs[2 * NSTREAM:]
    wa = wa_ref[...]
    wst = ws_ref[...].T
    batt = batt_ref[...] + abias_ref[...]
    bstrat = bstrat_ref[...]
    wc = jnp.concatenate([wa, wst], axis=1)
    for j in range(NSTREAM):
        x = x_refs[j][...]
        acc = jnp.dot(x, wc, preferred_element_type=jnp.float32)
        z = acc[:, :S] + batt + g_refs[j][...]
        sig = acc[:, S:] + bstrat
        out_ref[j] = jnp.concatenate([z, sig], axis=1).T


def _sc_body(zst_hbm, out_hbm, zv0, zv1, outv, sem0, sem1):
    wid = lax.axis_index("s") * 2 + lax.axis_index("c")
    bufs = (zv0, zv1)
    sems = (sem0, sem1)
    copies = []
    for k in range(CPW):
        chunk = wid * CPW + k
        copies.append(
            pltpu.async_copy(zst_hbm.at[chunk], bufs[k % 2], sems[k % 2]))
    for k in range(CPW):
        chunk = wid * CPW + k
        zv = bufs[k % 2]
        copies[k].wait()
        for g in range(NGRP):
            base = g * 16
            zs = [zv[s, pl.ds(base, 16)] for s in range(S)]
            m = zs[0]
            for s in range(1, S):
                m = jnp.maximum(m, zs[s])
            num = jnp.zeros((16,), jnp.float32)
            den = jnp.zeros((16,), jnp.float32)
            for s in range(S):
                e = jnp.exp(zs[s] - m)
                den = den + e
                num = num + e * zv[S + s, pl.ds(base, 16)]
            outv[pl.ds(base, 16)] = num / den
        pltpu.sync_copy(outv, out_hbm.at[pl.ds(chunk * T_TILE, T_TILE)])


@jax.jit
def kernel(x, gumbel_noise, W_att, b_att, W_strat, b_strat, adaptive_bias):
    ws = W_strat.reshape(S, D)
    batt = b_att.reshape(1, S)
    abias = adaptive_bias.reshape(1, S)
    bstrat = b_strat.reshape(1, S)
    grid = (T // (NSTREAM * T_TILE),)

    def xmap(j):
        return lambda i: (NSTREAM * i + j, 0)

    zst = pl.pallas_call(
        _tc_body,
        grid=grid,
        in_specs=(
            [pl.BlockSpec((T_TILE, D), xmap(j)) for j in range(NSTREAM)]
            + [pl.BlockSpec((T_TILE, S), xmap(j)) for j in range(NSTREAM)]
            + [
                pl.BlockSpec((D, S), lambda i: (0, 0)),
                pl.BlockSpec((S, D), lambda i: (0, 0)),
                pl.BlockSpec((1, S), lambda i: (0, 0)),
                pl.BlockSpec((1, S), lambda i: (0, 0)),
                pl.BlockSpec((1, S), lambda i: (0, 0)),
            ]
        ),
        out_specs=pl.BlockSpec((NSTREAM, 2 * S, T_TILE), lambda i: (i, 0, 0)),
        out_shape=jax.ShapeDtypeStruct((NCHUNK, 2 * S, T_TILE), jnp.float32),
    )(*([x] * NSTREAM + [gumbel_noise] * NSTREAM
        + [W_att, ws, batt, abias, bstrat]))

    gate = functools.partial(
        pl.kernel,
        mesh=plsc.VectorSubcoreMesh(core_axis_name="c", subcore_axis_name="s"),
        out_type=jax.ShapeDtypeStruct((T,), jnp.float32),
        scratch_types=[
            pltpu.VMEM((2 * S, T_TILE), jnp.float32),
            pltpu.VMEM((2 * S, T_TILE), jnp.float32),
            pltpu.VMEM((T_TILE,), jnp.float32),
            pltpu.SemaphoreType.DMA,
            pltpu.SemaphoreType.DMA,
        ],
    )(_sc_body)
    out = gate(zst)
    return out.reshape(T, 1)


# speedup vs baseline: 1.1409x; 1.1409x over previous
import functools

import jax
import jax.numpy as jnp
from jax import lax
from jax.experimental import pallas as pl
from jax.experimental.pallas import tpu as pltpu
from jax.experimental.pallas import tpu_sc as plsc

T, D, S = 16384, 2048, 16
T_TILE = 256
NSTREAM = 8
NCHUNK = T // T_TILE
NW = 32
CPW = NCHUNK // NW
NGRP = T_TILE // 16


def _tc_body(*refs):
    x_refs = refs[:NSTREAM]
    wat_ref, ws_ref, batt_ref, abias_ref, bstrat_ref, out_ref = refs[NSTREAM:]
    wa = wat_ref[...].T
    wst = ws_ref[...].T
    batt = batt_ref[...] + abias_ref[...]
    bstrat = bstrat_ref[...]
    wc = jnp.concatenate([wa, wst], axis=1)
    bc = jnp.concatenate([batt, bstrat], axis=1)
    for j in range(NSTREAM):
        x = x_refs[j][...]
        acc = jnp.dot(x, wc, preferred_element_type=jnp.float32) + bc
        out_ref[j] = acc.T


def _sc_body(zst_hbm, gt_hbm, out_hbm, zv0, zv1, gv0, gv1, outv,
             sem0, sem1, gsem0, gsem1):
    wid = lax.axis_index("s") * 2 + lax.axis_index("c")
    bufs = (zv0, zv1)
    gbufs = (gv0, gv1)
    sems = (sem0, sem1)
    gsems = (gsem0, gsem1)
    copies = []
    for k in range(CPW):
        chunk = wid * CPW + k
        copies.append(
            (pltpu.async_copy(zst_hbm.at[chunk], bufs[k % 2], sems[k % 2]),
             pltpu.async_copy(gt_hbm.at[:, pl.ds(chunk * T_TILE, T_TILE)],
                              gbufs[k % 2], gsems[k % 2])))
    for k in range(CPW):
        chunk = wid * CPW + k
        zv = bufs[k % 2]
        gv = gbufs[k % 2]
        copies[k][0].wait()
        copies[k][1].wait()
        for g in range(NGRP):
            base = g * 16
            zs = [zv[s, pl.ds(base, 16)] + gv[s, pl.ds(base, 16)]
                  for s in range(S)]
            m = zs[0]
            for s in range(1, S):
                m = jnp.maximum(m, zs[s])
            num = jnp.zeros((16,), jnp.float32)
            den = jnp.zeros((16,), jnp.float32)
            for s in range(S):
                e = jnp.exp(zs[s] - m)
                den = den + e
                num = num + e * zv[S + s, pl.ds(base, 16)]
            outv[pl.ds(base, 16)] = num / den
        pltpu.sync_copy(outv, out_hbm.at[pl.ds(chunk * T_TILE, T_TILE)])


@jax.jit
def kernel(x, gumbel_noise, W_att, b_att, W_strat, b_strat, adaptive_bias):
    wat = W_att.T
    ws = W_strat.reshape(S, D)
    gt = gumbel_noise.T
    batt = b_att.reshape(1, S)
    abias = adaptive_bias.reshape(1, S)
    bstrat = b_strat.reshape(1, S)
    grid = (T // (NSTREAM * T_TILE),)

    def xmap(j):
        return lambda i: (NSTREAM * i + j, 0)

    zst = pl.pallas_call(
        _tc_body,
        grid=grid,
        in_specs=(
            [pl.BlockSpec((T_TILE, D), xmap(j)) for j in range(NSTREAM)]
            + [
                pl.BlockSpec((S, D), lambda i: (0, 0)),
                pl.BlockSpec((S, D), lambda i: (0, 0)),
                pl.BlockSpec((1, S), lambda i: (0, 0)),
                pl.BlockSpec((1, S), lambda i: (0, 0)),
                pl.BlockSpec((1, S), lambda i: (0, 0)),
            ]
        ),
        out_specs=pl.BlockSpec((NSTREAM, 2 * S, T_TILE), lambda i: (i, 0, 0)),
        out_shape=jax.ShapeDtypeStruct((NCHUNK, 2 * S, T_TILE), jnp.float32),
    )(*([x] * NSTREAM + [wat, ws, batt, abias, bstrat]))

    gate = functools.partial(
        pl.kernel,
        mesh=plsc.VectorSubcoreMesh(core_axis_name="c", subcore_axis_name="s"),
        out_type=jax.ShapeDtypeStruct((T,), jnp.float32),
        scratch_types=[
            pltpu.VMEM((2 * S, T_TILE), jnp.float32),
            pltpu.VMEM((2 * S, T_TILE), jnp.float32),
            pltpu.VMEM((S, T_TILE), jnp.float32),
            pltpu.VMEM((S, T_TILE), jnp.float32),
            pltpu.VMEM((T_TILE,), jnp.float32),
            pltpu.SemaphoreType.DMA,
            pltpu.SemaphoreType.DMA,
            pltpu.SemaphoreType.DMA,
            pltpu.SemaphoreType.DMA,
        ],
    )(_sc_body)
    out = gate(zst, gt)
    return out.reshape(T, 1)


# speedup vs baseline: 1.6442x; 1.4411x over previous
import functools

import jax
import jax.numpy as jnp
from jax import lax
from jax.experimental import pallas as pl
from jax.experimental.pallas import tpu as pltpu

T, D, S = 16384, 2048, 16
T_TILE = 256
NSTREAM = 8
NCHUNK = T // T_TILE


def _tc_body(*refs):
    x_refs = refs[:NSTREAM]
    g_refs = refs[NSTREAM:2 * NSTREAM]
    wat_ref, ws_ref, batt_ref, abias_ref, bstrat_ref, out_ref = refs[2 * NSTREAM:]
    wa = wat_ref[...].T
    wst = ws_ref[...].T
    wc = jnp.concatenate([wa, wst], axis=1)
    battT = (batt_ref[...] + abias_ref[...]).T
    bstratT = bstrat_ref[...].T
    for j in range(NSTREAM):
        acc = jnp.dot(x_refs[j][...], wc, preferred_element_type=jnp.float32)
        accT = acc.T
        z = accT[:S, :] + battT + g_refs[j][...]
        m = jnp.max(z, axis=0, keepdims=True)
        e = jnp.exp(z - m)
        den = jnp.sum(e, axis=0, keepdims=True)
        sig = accT[S:, :] + bstratT
        num = jnp.sum(e * sig, axis=0, keepdims=True)
        out_ref[j:j + 1, :] = num / den


@jax.jit
def kernel(x, gumbel_noise, W_att, b_att, W_strat, b_strat, adaptive_bias):
    wat = W_att.T
    ws = W_strat.reshape(S, D)
    gt = gumbel_noise.T
    batt = b_att.reshape(1, S)
    abias = adaptive_bias.reshape(1, S)
    bstrat = b_strat.reshape(1, S)
    grid = (T // (NSTREAM * T_TILE),)

    def xmap(j):
        return lambda i: (NSTREAM * i + j, 0)

    def gmap(j):
        return lambda i: (0, NSTREAM * i + j)

    out = pl.pallas_call(
        _tc_body,
        grid=grid,
        in_specs=(
            [pl.BlockSpec((T_TILE, D), xmap(j)) for j in range(NSTREAM)]
            + [pl.BlockSpec((S, T_TILE), gmap(j)) for j in range(NSTREAM)]
            + [
                pl.BlockSpec((S, D), lambda i: (0, 0)),
                pl.BlockSpec((S, D), lambda i: (0, 0)),
                pl.BlockSpec((1, S), lambda i: (0, 0)),
                pl.BlockSpec((1, S), lambda i: (0, 0)),
                pl.BlockSpec((1, S), lambda i: (0, 0)),
            ]
        ),
        out_specs=pl.BlockSpec((NSTREAM, T_TILE), lambda i: (i, 0)),
        out_shape=jax.ShapeDtypeStruct((NCHUNK, T_TILE), jnp.float32),
    )(*([x] * NSTREAM + [gt] * NSTREAM + [wat, ws, batt, abias, bstrat]))
    return out.reshape(T, 1)


# speedup vs baseline: 1.6981x; 1.0328x over previous
import functools

import jax
import jax.numpy as jnp
from jax import lax
from jax.experimental import pallas as pl
from jax.experimental.pallas import tpu as pltpu

T, D, S = 16384, 2048, 16
T_TILE = 256
NSTREAM = 8
NCHUNK = T // T_TILE


def _tc_body(*refs):
    x_refs = refs[:NSTREAM]
    g_refs = refs[NSTREAM:2 * NSTREAM]
    wat_ref, ws_ref, batt_ref, abias_ref, bstrat_ref, out_ref = refs[2 * NSTREAM:]
    wa = wat_ref[...].T
    wst = ws_ref[...].reshape(S, D).T
    wc = jnp.concatenate([wa, wst], axis=1)
    battT = (batt_ref[...] + abias_ref[...]).T
    bstratT = bstrat_ref[...].T
    for j in range(NSTREAM):
        acc = jnp.dot(x_refs[j][...], wc, preferred_element_type=jnp.float32)
        accT = acc.T
        z = accT[:S, :] + battT + g_refs[j][...]
        m = jnp.max(z, axis=0, keepdims=True)
        e = jnp.exp(z - m)
        den = jnp.sum(e, axis=0, keepdims=True)
        sig = accT[S:, :] + bstratT
        num = jnp.sum(e * sig, axis=0, keepdims=True)
        out_ref[j:j + 1, :] = num / den


@jax.jit
def kernel(x, gumbel_noise, W_att, b_att, W_strat, b_strat, adaptive_bias):
    wat = W_att.T
    ws = W_strat.reshape(S * D)
    gt = gumbel_noise.T
    batt = b_att.reshape(1, S)
    abias = adaptive_bias.reshape(1, S)
    bstrat = b_strat.reshape(1, S)
    grid = (T // (NSTREAM * T_TILE),)

    def xmap(j):
        return lambda i: (NSTREAM * i + j, 0)

    def gmap(j):
        return lambda i: (0, NSTREAM * i + j)

    out = pl.pallas_call(
        _tc_body,
        grid=grid,
        in_specs=(
            [pl.BlockSpec((T_TILE, D), xmap(j)) for j in range(NSTREAM)]
            + [pl.BlockSpec((S, T_TILE), gmap(j)) for j in range(NSTREAM)]
            + [
                pl.BlockSpec((S, D), lambda i: (0, 0)),
                pl.BlockSpec((S * D,), lambda i: (0,)),
                pl.BlockSpec((1, S), lambda i: (0, 0)),
                pl.BlockSpec((1, S), lambda i: (0, 0)),
                pl.BlockSpec((1, S), lambda i: (0, 0)),
            ]
        ),
        out_specs=pl.BlockSpec((NSTREAM, T_TILE), lambda i: (i, 0)),
        out_shape=jax.ShapeDtypeStruct((NCHUNK, T_TILE), jnp.float32),
    )(*([x] * NSTREAM + [gt] * NSTREAM + [wat, ws, batt, abias, bstrat]))
    return out.reshape(T, 1)
